# baseline (speedup 1.0000x reference)
import functools

import jax
import jax.numpy as jnp
from jax import lax
from jax.experimental import pallas as pl
from jax.experimental.pallas import tpu as pltpu
from jax.experimental.pallas import tpu_sc as plsc

_MAX_LEN = 2048


def kernel(x, rel_pos_emb):
    batch, seq_len, d_model = x.shape
    info = plsc.get_sparse_core_info()
    nw = info.num_cores * info.num_subcores
    lanes = info.num_lanes
    s_per_w = seq_len // nw
    chunk = 16
    n_chunks = s_per_w // chunk
    groups = d_model // lanes
    n_rows = rel_pos_emb.shape[0]
    base = (_MAX_LEN - 1) // 8 * 8
    mesh = plsc.VectorSubcoreMesh(core_axis_name="c", subcore_axis_name="s")

    last_w = nw - 1
    last_win0 = base + last_w * s_per_w - 8
    tail_start = base + seq_len
    tail_len = n_rows - tail_start

    @functools.partial(
        pl.kernel,
        mesh=mesh,
        out_type=jax.ShapeDtypeStruct(x.shape, x.dtype),
        scratch_types=[
            pltpu.VMEM((s_per_w + 16, d_model), x.dtype),
            pltpu.VMEM((chunk, d_model), x.dtype),
        ],
    )
    def k(x_hbm, emb_hbm, out_hbm, emb_v, buf):
        wid = lax.axis_index("c") * info.num_subcores + lax.axis_index("s")
        s0 = wid * s_per_w

        @pl.when(wid != last_w)
        def _fetch_rows():
            win0 = pl.multiple_of(base + s0, 8)
            pltpu.sync_copy(
                emb_hbm.at[pl.ds(win0, s_per_w + 8), :],
                emb_v.at[pl.ds(0, s_per_w + 8), :],
            )

        @pl.when(wid == last_w)
        def _fetch_rows_last():
            pltpu.sync_copy(
                emb_hbm.at[pl.ds(last_win0, s_per_w + 8), :],
                emb_v.at[pl.ds(0, s_per_w + 8), :],
            )
            pltpu.sync_copy(
                emb_hbm.at[pl.ds(tail_start, tail_len), :],
                emb_v.at[pl.ds(s_per_w + 8, tail_len), :],
            )

        shift = lax.select(
            wid == last_w,
            _MAX_LEN - 1 + s0 - last_win0,
            (_MAX_LEN - 1) - base,
        )

        def outer(t, carry):
            b = t // n_chunks
            c = t % n_chunks
            pltpu.sync_copy(x_hbm.at[b, pl.ds(s0 + c * chunk, chunk), :], buf)

            def row(r, _):
                er = shift + c * chunk + r
                for g in range(groups):
                    sl = pl.ds(g * lanes, lanes)
                    buf[r, sl] = buf[r, sl] + emb_v[er, sl]
                return 0

            lax.fori_loop(0, chunk, row, 0)
            pltpu.sync_copy(buf, out_hbm.at[b, pl.ds(s0 + c * chunk, chunk), :])
            return carry

        lax.fori_loop(0, batch * n_chunks, outer, 0)

    return k(x, rel_pos_emb)


# speedup vs baseline: 1.3352x; 1.3352x over previous
import functools

import jax
import jax.numpy as jnp
from jax import lax
from jax.experimental import pallas as pl
from jax.experimental.pallas import tpu as pltpu
from jax.experimental.pallas import tpu_sc as plsc

_MAX_LEN = 2048


def kernel(x, rel_pos_emb):
    batch, seq_len, d_model = x.shape
    info = plsc.get_sparse_core_info()
    nw = info.num_cores * info.num_subcores
    lanes = info.num_lanes
    s_per_w = seq_len // nw
    chunk = 16
    n_chunks = s_per_w // chunk
    groups = d_model // lanes
    n_rows = rel_pos_emb.shape[0]
    base = (_MAX_LEN - 1) // 8 * 8
    mesh = plsc.VectorSubcoreMesh(core_axis_name="c", subcore_axis_name="s")

    last_w = nw - 1
    last_win0 = base + last_w * s_per_w - 8
    tail_start = base + seq_len
    tail_len = n_rows - tail_start

    @functools.partial(
        pl.kernel,
        mesh=mesh,
        out_type=jax.ShapeDtypeStruct(x.shape, x.dtype),
        scratch_types=[
            pltpu.VMEM((s_per_w + 16, d_model), x.dtype),
            pltpu.VMEM((chunk, d_model), x.dtype),
        ],
    )
    def k(x_hbm, emb_hbm, out_hbm, emb_v, buf):
        wid = lax.axis_index("c") * info.num_subcores + lax.axis_index("s")
        s0 = wid * s_per_w

        @pl.when(wid != last_w)
        def _fetch_rows():
            win0 = pl.multiple_of(base + s0, 8)
            pltpu.sync_copy(
                emb_hbm.at[pl.ds(win0, s_per_w + 8), :],
                emb_v.at[pl.ds(0, s_per_w + 8), :],
            )

        @pl.when(wid == last_w)
        def _fetch_rows_last():
            pltpu.sync_copy(
                emb_hbm.at[pl.ds(last_win0, s_per_w + 8), :],
                emb_v.at[pl.ds(0, s_per_w + 8), :],
            )
            pltpu.sync_copy(
                emb_hbm.at[pl.ds(tail_start, tail_len), :],
                emb_v.at[pl.ds(s_per_w + 8, tail_len), :],
            )

        shift = lax.select(
            wid == last_w,
            _MAX_LEN - 1 + s0 - last_win0,
            (_MAX_LEN - 1) - base,
        )

        def outer(t, carry):
            b = t // n_chunks
            c = t % n_chunks
            pltpu.sync_copy(x_hbm.at[b, pl.ds(s0 + c * chunk, chunk), :], buf)

            @plsc.parallel_loop(0, chunk, 1, unroll=2)
            def row(r):
                er = shift + c * chunk + r
                for g in range(groups):
                    sl = pl.ds(g * lanes, lanes)
                    buf[r, sl] = buf[r, sl] + emb_v[er, sl]
            pltpu.sync_copy(buf, out_hbm.at[b, pl.ds(s0 + c * chunk, chunk), :])
            return carry

        lax.fori_loop(0, batch * n_chunks, outer, 0)

    return k(x, rel_pos_emb)


# speedup vs baseline: 1.4459x; 1.0829x over previous
import functools

import jax
import jax.numpy as jnp
from jax import lax
from jax.experimental import pallas as pl
from jax.experimental.pallas import tpu as pltpu
from jax.experimental.pallas import tpu_sc as plsc

_MAX_LEN = 2048


def kernel(x, rel_pos_emb):
    batch, seq_len, d_model = x.shape
    info = plsc.get_sparse_core_info()
    nw = info.num_cores * info.num_subcores
    lanes = info.num_lanes
    s_per_w = seq_len // nw
    chunk = 32
    n_chunks = s_per_w // chunk
    groups = d_model // lanes
    n_rows = rel_pos_emb.shape[0]
    base = (_MAX_LEN - 1) // 8 * 8
    mesh = plsc.VectorSubcoreMesh(core_axis_name="c", subcore_axis_name="s")

    last_w = nw - 1
    last_win0 = base + last_w * s_per_w - 8
    tail_start = base + seq_len
    tail_len = n_rows - tail_start

    @functools.partial(
        pl.kernel,
        mesh=mesh,
        out_type=jax.ShapeDtypeStruct(x.shape, x.dtype),
        scratch_types=[
            pltpu.VMEM((s_per_w + 16, d_model), x.dtype),
            pltpu.VMEM((chunk, d_model), x.dtype),
        ],
    )
    def k(x_hbm, emb_hbm, out_hbm, emb_v, buf):
        wid = lax.axis_index("c") * info.num_subcores + lax.axis_index("s")
        s0 = wid * s_per_w

        @pl.when(wid != last_w)
        def _fetch_rows():
            win0 = pl.multiple_of(base + s0, 8)
            pltpu.sync_copy(
                emb_hbm.at[pl.ds(win0, s_per_w + 8), :],
                emb_v.at[pl.ds(0, s_per_w + 8), :],
            )

        @pl.when(wid == last_w)
        def _fetch_rows_last():
            pltpu.sync_copy(
                emb_hbm.at[pl.ds(last_win0, s_per_w + 8), :],
                emb_v.at[pl.ds(0, s_per_w + 8), :],
            )
            pltpu.sync_copy(
                emb_hbm.at[pl.ds(tail_start, tail_len), :],
                emb_v.at[pl.ds(s_per_w + 8, tail_len), :],
            )

        shift = lax.select(
            wid == last_w,
            _MAX_LEN - 1 + s0 - last_win0,
            (_MAX_LEN - 1) - base,
        )

        def outer(t, carry):
            b = t // n_chunks
            c = t % n_chunks
            pltpu.sync_copy(x_hbm.at[b, pl.ds(s0 + c * chunk, chunk), :], buf)

            @plsc.parallel_loop(0, chunk, 1, unroll=4)
            def row(r):
                er = shift + c * chunk + r
                for g in range(groups):
                    sl = pl.ds(g * lanes, lanes)
                    buf[r, sl] = buf[r, sl] + emb_v[er, sl]
            pltpu.sync_copy(buf, out_hbm.at[b, pl.ds(s0 + c * chunk, chunk), :])
            return carry

        lax.fori_loop(0, batch * n_chunks, outer, 0)

    return k(x, rel_pos_emb)


# speedup vs baseline: 5.3301x; 3.6864x over previous
import functools

import jax
import jax.numpy as jnp
from jax.experimental import pallas as pl
from jax.experimental.pallas import tpu as pltpu

_MAX_LEN = 2048


def _half(x_ref, o_ref, emb_vmem, h0, hrows, shift):
    win = emb_vmem[pl.ds(h0, hrows + 8), :]
    rows = jax.lax.slice(win, (shift, 0), (shift + hrows, win.shape[1]))
    o_ref[0, pl.ds(h0, hrows), :] = x_ref[0, pl.ds(h0, hrows), :] + rows


def _body(x_ref, emb_hbm, o_ref, emb_vmem, tail_vmem, sem_a, sem_b, sem_t,
          *, seq_len, base, shift):
    b = pl.program_id(1)
    half = seq_len // 2
    a_rows = half + 8
    b_rows = seq_len - a_rows

    @pl.when(b == 0)
    def _start_dmas():
        pltpu.make_async_copy(
            emb_hbm.at[pl.ds(base, a_rows), :],
            emb_vmem.at[pl.ds(0, a_rows), :],
            sem_a,
        ).start()
        pltpu.make_async_copy(
            emb_hbm.at[pl.ds(base + a_rows, b_rows), :],
            emb_vmem.at[pl.ds(a_rows, b_rows), :],
            sem_b,
        ).start()
        pltpu.make_async_copy(
            emb_hbm.at[pl.ds(base + seq_len, shift), :], tail_vmem, sem_t
        ).start()
        pltpu.make_async_copy(
            emb_hbm.at[pl.ds(base, a_rows), :],
            emb_vmem.at[pl.ds(0, a_rows), :],
            sem_a,
        ).wait()

    _half(x_ref, o_ref, emb_vmem, 0, half, shift)

    @pl.when(b == 0)
    def _wait_rest():
        pltpu.make_async_copy(
            emb_hbm.at[pl.ds(base + a_rows, b_rows), :],
            emb_vmem.at[pl.ds(a_rows, b_rows), :],
            sem_b,
        ).wait()
        pltpu.make_async_copy(
            emb_hbm.at[pl.ds(base + seq_len, shift), :], tail_vmem, sem_t
        ).wait()
        pad = jnp.zeros((8 - shift, tail_vmem.shape[1]), tail_vmem.dtype)
        emb_vmem[pl.ds(seq_len, 8), :] = jnp.concatenate(
            [tail_vmem[...], pad], axis=0
        )

    _half(x_ref, o_ref, emb_vmem, half, half, shift)


def kernel(x, rel_pos_emb):
    batch, seq_len, d_model = x.shape
    base = (_MAX_LEN - 1) // 8 * 8
    shift = (_MAX_LEN - 1) - base
    body = functools.partial(_body, seq_len=seq_len, base=base, shift=shift)
    return pl.pallas_call(
        body,
        grid=(1, batch),
        in_specs=[
            pl.BlockSpec((1, seq_len, d_model), lambda i, b: (b, i, 0)),
            pl.BlockSpec(memory_space=pltpu.MemorySpace.HBM),
        ],
        out_specs=pl.BlockSpec((1, seq_len, d_model), lambda i, b: (b, i, 0)),
        out_shape=jax.ShapeDtypeStruct(x.shape, x.dtype),
        scratch_shapes=[
            pltpu.VMEM((seq_len + 8, d_model), x.dtype),
            pltpu.VMEM((shift, d_model), x.dtype),
            pltpu.SemaphoreType.DMA,
            pltpu.SemaphoreType.DMA,
            pltpu.SemaphoreType.DMA,
        ],
    )(x, rel_pos_emb)
